# baseline (speedup 1.0000x reference)
import functools

import jax
import jax.numpy as jnp
from jax import lax
from jax.experimental import pallas as pl
from jax.experimental.pallas import tpu as pltpu
from jax.experimental.pallas import tpu_sc as plsc

G = 4
NC = 2
NS = 16
NW = NC * NS


def _make_gather(n_embed: int, d: int, batch: int, hist: int):
    assert batch % (NW * 2 * G) == 0
    rows_per_w = batch // NW
    n_pairs = rows_per_w // (2 * G)

    mesh = plsc.VectorSubcoreMesh(core_axis_name="c", subcore_axis_name="s")

    @functools.partial(
        pl.kernel,
        mesh=mesh,
        compiler_params=pltpu.CompilerParams(use_tc_tiling_on_sc=False),
        out_type=jax.ShapeDtypeStruct((batch, hist, d), jnp.float32),
        scratch_types=[
            pltpu.VMEM((rows_per_w, hist), jnp.int32),
            pltpu.VMEM((G, hist, 128), jnp.float32),
            pltpu.VMEM((G, hist, 128), jnp.float32),
            pltpu.SemaphoreType.DMA,
            pltpu.SemaphoreType.DMA,
            pltpu.SemaphoreType.DMA,
            pltpu.SemaphoreType.DMA,
        ],
    )
    def gather(idx_hbm, table_hbm, out_hbm,
               idx_v, rows_a, rows_b, gsem_a, gsem_b, wsem_a, wsem_b):
        wid = lax.axis_index("s") * NC + lax.axis_index("c")
        row_base = wid * rows_per_w
        pltpu.sync_copy(idx_hbm.at[pl.ds(row_base, rows_per_w)], idx_v)

        def fire(g, buf, sem):
            return [
                pltpu.async_copy(
                    table_hbm.at[idx_v.at[g * G + b]],
                    buf.at[b],
                    sem,
                )
                for b in range(G)
            ]

        def write(g, buf, sem):
            row0 = row_base + g * G
            return pltpu.async_copy(
                buf.at[:, :, pl.ds(0, d)],
                out_hbm.at[pl.ds(row0, G)],
                sem,
            )

        def pair_body(k, carry):
            g0 = 2 * k
            ha = fire(g0, rows_a, gsem_a)
            hb = fire(g0 + 1, rows_b, gsem_b)
            for h in ha:
                h.wait()
            wa = write(g0, rows_a, wsem_a)
            for h in hb:
                h.wait()
            wb = write(g0 + 1, rows_b, wsem_b)
            wa.wait()
            wb.wait()
            return carry

        lax.fori_loop(0, n_pairs, pair_body, 0)

    return gather


def kernel(inputs, lookup_table):
    batch, hist = inputs.shape
    n_embed, d = lookup_table.shape
    idx = inputs if inputs.dtype == jnp.int32 else inputs.astype(jnp.int32)
    padded = jnp.pad(lookup_table, ((0, 0), (0, 128 - d)))
    return _make_gather(n_embed, d, batch, hist)(idx, padded)
